# baseline (speedup 1.0000x reference)
import functools

import jax
import jax.numpy as jnp
import numpy as np
from jax import lax
from jax.experimental import pallas as pl
from jax.experimental.pallas import tpu as pltpu
from jax.experimental.pallas import tpu_sc as plsc

VOCAB = 32000
HIDDEN = 768
BATCH = 4096
SEQ = 200

NC = 2
NS = 16
NW = NC * NS

NCHUNK = 5
CHUNK = SEQ // NCHUNK
HGRP = HIDDEN // 32
HW = HIDDEN // 2
SCALE = 1.0 / SEQ

KCH = 4
CB = BATCH // KCH
B_PER_W = CB // NW

_MASK_HI = np.uint32(0xFFFF0000)
_SIXTEEN = np.uint32(16)
_ONE = np.uint32(1)
_HALF = np.uint32(0x7FFF)

_mesh = plsc.VectorSubcoreMesh(core_axis_name="c", subcore_axis_name="s")


def _split(x32):
    u = plsc.bitcast(x32, jnp.uint32)
    even = plsc.bitcast(u << _SIXTEEN, jnp.float32)
    odd = plsc.bitcast(u & _MASK_HI, jnp.float32)
    return even, odd


def _pack_rtne(e, o):
    ue = plsc.bitcast(e, jnp.uint32)
    uo = plsc.bitcast(o, jnp.uint32)
    ue = ue + (_HALF + ((ue >> _SIXTEEN) & _ONE))
    uo = uo + (_HALF + ((uo >> _SIXTEEN) & _ONE))
    return (ue >> _SIXTEEN) | (uo & _MASK_HI)


@functools.partial(
    pl.kernel,
    out_type=jax.ShapeDtypeStruct((CB * HW,), jnp.uint32),
    mesh=_mesh,
    compiler_params=pltpu.CompilerParams(needs_layout_passes=False),
    scratch_types=[
        pltpu.VMEM((B_PER_W * SEQ,), jnp.int32),
        pltpu.VMEM((2, CHUNK, HW), jnp.uint32),
        pltpu.VMEM((HIDDEN,), jnp.float32),
        pltpu.VMEM((HW,), jnp.uint32),
        pltpu.SemaphoreType.DMA,
        pltpu.SemaphoreType.DMA,
    ],
)
def _pool_kernel(ids_hbm, table_hbm, out_hbm, idx_v, rows_v, acc_v, obuf_v,
                 sem0, sem1):
    wid = lax.axis_index("s") * NC + lax.axis_index("c")
    base = wid * B_PER_W
    sems = (sem0, sem1)

    pltpu.sync_copy(ids_hbm.at[pl.ds(base * SEQ, B_PER_W * SEQ)], idx_v)

    def accum(buf, c):
        def h_body(h, _):
            hh = pl.multiple_of(h * 16, 16)
            sl = pl.ds(hh, 16)
            ae = [None] * 4
            ao = [None] * 4
            for j in range(4):
                ae[j], ao[j] = _split(rows_v[buf, j, sl])
            for k in range(4, CHUNK, 4):
                for j in range(4):
                    e, o = _split(rows_v[buf, k + j, sl])
                    ae[j] = ae[j] + e
                    ao[j] = ao[j] + o
            sum_e = (ae[0] + ae[1]) + (ae[2] + ae[3])
            sum_o = (ao[0] + ao[1]) + (ao[2] + ao[3])
            hh2 = pl.multiple_of(h * 32, 32)
            esl = pl.ds(hh2, 16)
            osl = pl.ds(hh2 + 16, 16)
            if c == 0:
                acc_v[esl] = sum_e
                acc_v[osl] = sum_o
            elif c < NCHUNK - 1:
                acc_v[esl] = acc_v[esl] + sum_e
                acc_v[osl] = acc_v[osl] + sum_o
            else:
                tot_e = (acc_v[esl] + sum_e) * SCALE
                tot_o = (acc_v[osl] + sum_o) * SCALE
                obuf_v[sl] = _pack_rtne(tot_e, tot_o)
            return 0

        lax.fori_loop(0, HGRP, h_body, 0)

    def idx_slice(r, c):
        off = pl.multiple_of(r * SEQ + c * CHUNK, CHUNK)
        return idx_v.at[pl.ds(off, CHUNK)]

    def row_body(r, _):
        b = base + r
        copies = [None, None]
        copies[0] = pltpu.async_copy(
            table_hbm.at[idx_slice(r, 0)], rows_v.at[0], sems[0])
        for c in range(NCHUNK):
            nxt = (c + 1) % 2
            if c + 1 < NCHUNK:
                copies[nxt] = pltpu.async_copy(
                    table_hbm.at[idx_slice(r, c + 1)], rows_v.at[nxt], sems[nxt])
            copies[c % 2].wait()
            accum(c % 2, c)
        off = pl.multiple_of(b * HW, HW)
        pltpu.sync_copy(obuf_v, out_hbm.at[pl.ds(off, HW)])
        return 0

    lax.fori_loop(0, B_PER_W, row_body, 0)


def _mm_body(x_ref, w_ref, b_ref, o_ref):
    o_ref[...] = (
        jnp.dot(x_ref[...], w_ref[...], preferred_element_type=jnp.float32)
        + b_ref[...]
    )


def _matmul(pooled, Wb, b2d):
    M, K = pooled.shape
    N = Wb.shape[1]
    BN = 3200
    return pl.pallas_call(
        _mm_body,
        grid=(N // BN,),
        in_specs=[
            pl.BlockSpec((M, K), lambda j: (0, 0)),
            pl.BlockSpec((K, BN), lambda j: (0, j)),
            pl.BlockSpec((1, BN), lambda j: (0, j)),
        ],
        out_specs=pl.BlockSpec((M, BN), lambda j: (0, j)),
        out_shape=jax.ShapeDtypeStruct((M, N), jnp.float32),
    )(pooled, Wb, b2d)


@jax.jit
def kernel(input_ids, embedding, W, b):
    ids = input_ids.astype(jnp.int32).reshape(BATCH * SEQ)
    table = jax.lax.bitcast_convert_type(
        embedding.astype(jnp.bfloat16).reshape(VOCAB, HW, 2), jnp.uint32)
    Wb = W.astype(jnp.bfloat16)
    b2d = b.reshape(1, VOCAB)
    outs = []
    for k in range(KCH):
        bits = _pool_kernel(
            lax.dynamic_slice(ids, (k * CB * SEQ,), (CB * SEQ,)), table)
        pooled = jax.lax.bitcast_convert_type(
            bits.reshape(CB, HW), jnp.bfloat16).reshape(CB, HIDDEN)
        outs.append(_matmul(pooled, Wb, b2d))
    return jnp.concatenate(outs, axis=0)


# speedup vs baseline: 1.6533x; 1.6533x over previous
import functools

import jax
import jax.numpy as jnp
import numpy as np
from jax import lax
from jax.experimental import pallas as pl
from jax.experimental.pallas import tpu as pltpu
from jax.experimental.pallas import tpu_sc as plsc

VOCAB = 32000
HIDDEN = 768
BATCH = 4096
SEQ = 200

NC = 2
NS = 16
NW = NC * NS
B_PER_W = BATCH // NW

NCHUNK = 5
CHUNK = SEQ // NCHUNK
HW = HIDDEN // 2
HGRP = HW // 16
SCALE = 1.0 / SEQ

_MASK_HI = np.uint32(0xFFFF0000)
_SIXTEEN = np.uint32(16)
_ONE = np.uint32(1)
_HALF = np.uint32(0x7FFF)

_mesh = plsc.VectorSubcoreMesh(core_axis_name="c", subcore_axis_name="s")


def _split(x32):
    u = plsc.bitcast(x32, jnp.uint32)
    lo = plsc.bitcast(u << _SIXTEEN, jnp.float32)
    hi = plsc.bitcast(u & _MASK_HI, jnp.float32)
    return lo, hi


@functools.partial(
    pl.kernel,
    out_type=jax.ShapeDtypeStruct((BATCH * HIDDEN,), jnp.float32),
    mesh=_mesh,
    compiler_params=pltpu.CompilerParams(needs_layout_passes=False),
    scratch_types=[
        pltpu.VMEM((B_PER_W * SEQ,), jnp.int32),
        pltpu.VMEM((2, CHUNK, HW), jnp.uint32),
        pltpu.VMEM((HIDDEN,), jnp.float32),
        pltpu.SemaphoreType.DMA,
        pltpu.SemaphoreType.DMA,
    ],
)
def _pool_kernel(ids_hbm, table_hbm, out_hbm, idx_v, rows_v, acc_v, sem0, sem1):
    wid = lax.axis_index("s") * NC + lax.axis_index("c")
    base = wid * B_PER_W
    sems = (sem0, sem1)

    pltpu.sync_copy(ids_hbm.at[pl.ds(base * SEQ, B_PER_W * SEQ)], idx_v)

    def accum(buf, c):
        def h_body(h, _):
            hh = pl.multiple_of(h * 16, 16)
            sl = pl.ds(hh, 16)
            al = [None] * 4
            ah = [None] * 4
            for j in range(4):
                al[j], ah[j] = _split(rows_v[buf, j, sl])
            for k in range(4, CHUNK, 4):
                for j in range(4):
                    lo, hi = _split(rows_v[buf, k + j, sl])
                    al[j] = al[j] + lo
                    ah[j] = ah[j] + hi
            sum_l = (al[0] + al[1]) + (al[2] + al[3])
            sum_h = (ah[0] + ah[1]) + (ah[2] + ah[3])
            lsl = pl.ds(hh, 16)
            hsl = pl.ds(hh + HW, 16)
            if c == 0:
                acc_v[lsl] = sum_l
                acc_v[hsl] = sum_h
            elif c < NCHUNK - 1:
                acc_v[lsl] = acc_v[lsl] + sum_l
                acc_v[hsl] = acc_v[hsl] + sum_h
            else:
                acc_v[lsl] = (acc_v[lsl] + sum_l) * SCALE
                acc_v[hsl] = (acc_v[hsl] + sum_h) * SCALE
            return 0

        lax.fori_loop(0, HGRP, h_body, 0)

    def idx_slice(r, c):
        off = pl.multiple_of(r * SEQ + c * CHUNK, CHUNK)
        return idx_v.at[pl.ds(off, CHUNK)]

    def row_body(r, _):
        b = base + r
        copies = [None, None]
        copies[0] = pltpu.async_copy(
            table_hbm.at[idx_slice(r, 0)], rows_v.at[0], sems[0])
        for c in range(NCHUNK):
            nxt = (c + 1) % 2
            if c + 1 < NCHUNK:
                copies[nxt] = pltpu.async_copy(
                    table_hbm.at[idx_slice(r, c + 1)], rows_v.at[nxt], sems[nxt])
            copies[c % 2].wait()
            accum(c % 2, c)
        off = pl.multiple_of(b * HIDDEN, HIDDEN)
        pltpu.sync_copy(acc_v, out_hbm.at[pl.ds(off, HIDDEN)])
        return 0

    lax.fori_loop(0, B_PER_W, row_body, 0)


def _pack_body(e_ref, o_ref):
    bits = jax.lax.bitcast_convert_type(e_ref[...], jnp.uint32)
    lo = bits[:, :HW]
    hi = bits[:, HW:]
    lo = lo + (_HALF + ((lo >> _SIXTEEN) & _ONE))
    hi = hi + (_HALF + ((hi >> _SIXTEEN) & _ONE))
    o_ref[...] = (lo >> _SIXTEEN) | (hi & _MASK_HI)


def _pack_table(embedding):
    BV = 2000
    return pl.pallas_call(
        _pack_body,
        grid=(VOCAB // BV,),
        in_specs=[pl.BlockSpec((BV, HIDDEN), lambda i: (i, 0))],
        out_specs=pl.BlockSpec((BV, HW), lambda i: (i, 0)),
        out_shape=jax.ShapeDtypeStruct((VOCAB, HW), jnp.uint32),
    )(embedding)


def _mm_body(x_ref, w_ref, b_ref, o_ref):
    o_ref[...] = (
        jnp.dot(x_ref[...].astype(jnp.bfloat16), w_ref[...],
                preferred_element_type=jnp.float32)
        + b_ref[...]
    )


def _matmul(pooled, Wb, b2d):
    M, K = pooled.shape
    N = Wb.shape[1]
    BM = 2048
    BN = 1280
    return pl.pallas_call(
        _mm_body,
        grid=(M // BM, N // BN),
        in_specs=[
            pl.BlockSpec((BM, K), lambda i, j: (i, 0)),
            pl.BlockSpec((K, BN), lambda i, j: (0, j)),
            pl.BlockSpec((1, BN), lambda i, j: (0, j)),
        ],
        out_specs=pl.BlockSpec((BM, BN), lambda i, j: (i, j)),
        out_shape=jax.ShapeDtypeStruct((M, N), jnp.float32),
    )(pooled, Wb, b2d)


@jax.jit
def kernel(input_ids, embedding, W, b):
    ids = input_ids.astype(jnp.int32).reshape(BATCH * SEQ)
    table = _pack_table(embedding)
    Wb = W.astype(jnp.bfloat16)
    pooled = _pool_kernel(ids, table).reshape(BATCH, HIDDEN)
    return _matmul(pooled, Wb, b.reshape(1, VOCAB))
